# baseline (speedup 1.0000x reference)
import functools

import numpy as np
import jax
import jax.numpy as jnp
from jax import lax
from jax.experimental import pallas as pl
from jax.experimental.pallas import tpu as pltpu
from jax.experimental.pallas import tpu_sc as plsc

N = 10000
E = 320000
D = 128
NP = 10240
NC = 2
NS = 16
NW = NC * NS
L = 16
C = 128
WPW = 80
NWIN = NW * WPW
HALF = WPW // 2
ROWS_PER_SUB = NP // NS

_PAD_AR = np.arange(NWIN * C - E, dtype=np.int32)
_PADS = np.stack([_PAD_AR % N, N + _PAD_AR % (NP - N)])


def _make_agg_kernel(with_cnt):
  mesh = plsc.VectorSubcoreMesh(core_axis_name="c", subcore_axis_name="s")

  out_type = [jax.ShapeDtypeStruct((NC, NP, D), jnp.float32)]
  if with_cnt:
    out_type.append(jax.ShapeDtypeStruct((NC, NP), jnp.float32))

  scratch = [
      pltpu.VMEM((HALF, C), jnp.int32),
      pltpu.VMEM((HALF, C), jnp.int32),
      pltpu.VMEM((C, D), jnp.float32),
      pltpu.VMEM((C, D), jnp.float32),
      pltpu.VMEM((C,), jnp.float32),
      pltpu.VMEM((ROWS_PER_SUB,), jnp.float32),
      pltpu.VMEM_SHARED((NP, D), jnp.float32),
      pltpu.VMEM_SHARED((NP,), jnp.float32),
      pltpu.SemaphoreType.DMA,
      pltpu.SemaphoreType.DMA,
  ]

  @functools.partial(pl.kernel, out_type=out_type, mesh=mesh,
                     scratch_types=scratch)
  def agg(edge_hbm, feat_hbm, *refs):
    if with_cnt:
      agg_out, cnt_out = refs[0], refs[1]
      refs = refs[2:]
    else:
      agg_out = refs[0]
      refs = refs[1:]
    (src_v, dst_v, rows_a, rows_b, ones_v, cbuf_v,
     agg_sh, cnt_sh, sem_a, sem_b) = refs

    c = lax.axis_index("c")
    s = lax.axis_index("s")
    wid = s * NC + c

    zf = jnp.zeros((L,), jnp.float32)
    of = jnp.ones((L,), jnp.float32)

    def zfill_step(j, carry):
      for g in range(D // L):
        rows_a[j, pl.ds(g * L, L)] = zf
      return carry
    lax.fori_loop(0, C, zfill_step, 0)
    if with_cnt:
      for i in range(C // L):
        ones_v[pl.ds(i * L, L)] = of
      for i in range(ROWS_PER_SUB // L):
        cbuf_v[pl.ds(i * L, L)] = zf

    row0 = s * ROWS_PER_SUB

    NZ = ROWS_PER_SUB // C
    for j in range(NZ):
      pltpu.sync_copy(rows_a, agg_sh.at[pl.ds(row0 + j * C, C)])
    if with_cnt:
      pltpu.sync_copy(cbuf_v, cnt_sh.at[pl.ds(row0, ROWS_PER_SUB)])
    plsc.subcore_barrier()

    def gather(g, buf, sem):
      pltpu.async_copy(feat_hbm.at[src_v.at[g]], buf, sem)

    def gwait(buf, sem):
      pltpu.make_async_copy(feat_hbm.at[src_v.at[0]], buf, sem).wait()

    def scat(g, buf):
      pltpu.sync_copy(buf, agg_sh.at[dst_v.at[g]], add=True)
      if with_cnt:
        pltpu.sync_copy(ones_v, cnt_sh.at[dst_v.at[g]], add=True)

    for half in range(2):
      w0 = wid * WPW + half * HALF
      pltpu.sync_copy(edge_hbm.at[0, pl.ds(w0, HALF)], src_v)
      pltpu.sync_copy(edge_hbm.at[1, pl.ds(w0, HALF)], dst_v)
      gather(0, rows_a, sem_a)

      def pair_step(k, carry):
        g0 = 2 * k
        gather(g0 + 1, rows_b, sem_b)
        gwait(rows_a, sem_a)
        scat(g0, rows_a)

        @pl.when(k < HALF // 2 - 1)
        def _():
          gather(g0 + 2, rows_a, sem_a)
        gwait(rows_b, sem_b)
        scat(g0 + 1, rows_b)
        return carry
      lax.fori_loop(0, HALF // 2, pair_step, 0)
    plsc.subcore_barrier()

    nwaits = {id(sem_a): 0, id(sem_b): 0}
    for j in range(ROWS_PER_SUB // C):
      buf, sem = (rows_a, sem_a) if j % 2 == 0 else (rows_b, sem_b)
      if j >= 2:
        pltpu.make_async_copy(buf, agg_out.at[c, pl.ds(row0, C)], sem).wait()
        nwaits[id(sem)] += 1
      r = row0 + j * C
      pltpu.sync_copy(agg_sh.at[pl.ds(r, C)], buf)
      pltpu.async_copy(buf, agg_out.at[c, pl.ds(r, C)], sem)
    if with_cnt:
      pltpu.sync_copy(cnt_sh.at[pl.ds(row0, ROWS_PER_SUB)], cbuf_v)
      pltpu.sync_copy(cbuf_v, cnt_out.at[c, pl.ds(row0, ROWS_PER_SUB)])
    for j in range(ROWS_PER_SUB // C):
      buf, sem = (rows_a, sem_a) if j % 2 == 0 else (rows_b, sem_b)
      if nwaits[id(sem)] > 0:
        nwaits[id(sem)] -= 1
        continue
      pltpu.make_async_copy(buf, agg_out.at[c, pl.ds(row0, C)], sem).wait()

  return agg


_agg_cnt = _make_agg_kernel(True)
_agg_nocnt = _make_agg_kernel(False)

BR = 1000


def _dot_t(a, w):
  return lax.dot_general(a, w, (((1,), (1,)), ((), ())),
                         preferred_element_type=jnp.float32)


def _self_body(x_ref, wr_ref, b_ref, o_ref):
  o_ref[...] = _dot_t(x_ref[...], wr_ref[...]) + b_ref[...]


def _self_term(x, wr, b):
  return pl.pallas_call(
      _self_body,
      grid=(N // BR,),
      in_specs=[
          pl.BlockSpec((BR, D), lambda i: (i, 0)),
          pl.BlockSpec((D, D), lambda i: (0, 0)),
          pl.BlockSpec((1, D), lambda i: (0, 0)),
      ],
      out_specs=pl.BlockSpec((BR, D), lambda i: (i, 0)),
      out_shape=jax.ShapeDtypeStruct((N, D), jnp.float32),
  )(x, wr, b)


def _combine_body(relu, aggs_ref, cnt_ref, self_ref, wl_ref, o_ref):
  cnt = cnt_ref[0] + cnt_ref[1]
  inv = 1.0 / jnp.maximum(cnt, 1.0)
  mean = (aggs_ref[0] + aggs_ref[1]) * inv
  out = _dot_t(mean, wl_ref[...]) + self_ref[...]
  if relu:
    out = jnp.maximum(out, 0.0)
  o_ref[...] = out


def _combine(aggs, cnts3, selfterm, wl, relu):
  return pl.pallas_call(
      functools.partial(_combine_body, relu),
      grid=(N // BR,),
      in_specs=[
          pl.BlockSpec((NC, BR, D), lambda i: (0, i, 0)),
          pl.BlockSpec((NC, BR, 1), lambda i: (0, i, 0)),
          pl.BlockSpec((BR, D), lambda i: (i, 0)),
          pl.BlockSpec((D, D), lambda i: (0, 0)),
      ],
      out_specs=pl.BlockSpec((BR, D), lambda i: (i, 0)),
      out_shape=jax.ShapeDtypeStruct((N, D), jnp.float32),
  )(aggs, cnts3, selfterm, wl)


def kernel(x, edge_index, W1_l, b1_l, W1_r, W2_l, b2_l, W2_r):
  edge2d = jnp.concatenate(
      [edge_index.astype(jnp.int32), jnp.asarray(_PADS)], axis=1
  ).reshape(2, NWIN, C)

  aggs1, cnts = _agg_cnt(edge2d, x)
  cnts3 = cnts.reshape(NC, NP, 1)
  self1 = _self_term(x, W1_r, b1_l.reshape(1, D))
  h = _combine(aggs1, cnts3, self1, W1_l, True)
  (aggs2,) = _agg_nocnt(edge2d, h)
  self2 = _self_term(h, W2_r, b2_l.reshape(1, D))
  return _combine(aggs2, cnts3, self2, W2_l, False)


# speedup vs baseline: 1.0082x; 1.0082x over previous
import functools

import numpy as np
import jax
import jax.numpy as jnp
from jax import lax
from jax.experimental import pallas as pl
from jax.experimental.pallas import tpu as pltpu
from jax.experimental.pallas import tpu_sc as plsc

N = 10000
E = 320000
D = 128
NP = 10240
NC = 2
NS = 16
NW = NC * NS
L = 16
C = 128
WPW = 80
NWIN = NW * WPW
HALF = WPW // 2
ROWS_PER_SUB = NP // NS

_PAD_AR = np.arange(NWIN * C - E, dtype=np.int32)
_PADS = np.stack([_PAD_AR % N, N + _PAD_AR % (NP - N)])


def _make_agg_kernel(with_cnt):
  mesh = plsc.VectorSubcoreMesh(core_axis_name="c", subcore_axis_name="s")

  out_type = [jax.ShapeDtypeStruct((NC, NP, D), jnp.float32)]
  if with_cnt:
    out_type.append(jax.ShapeDtypeStruct((NC, NP), jnp.float32))

  scratch = [
      pltpu.VMEM((HALF, C), jnp.int32),
      pltpu.VMEM((HALF, C), jnp.int32),
      pltpu.VMEM((C, D), jnp.float32),
      pltpu.VMEM((C, D), jnp.float32),
      pltpu.VMEM((C,), jnp.float32),
      pltpu.VMEM((16, D), jnp.float32),
      pltpu.VMEM((ROWS_PER_SUB,), jnp.float32),
      pltpu.VMEM_SHARED((NP, D), jnp.float32),
      pltpu.VMEM_SHARED((NP,), jnp.float32),
      pltpu.SemaphoreType.DMA,
      pltpu.SemaphoreType.DMA,
  ]

  @functools.partial(pl.kernel, out_type=out_type, mesh=mesh,
                     scratch_types=scratch)
  def agg(edge_hbm, feat_hbm, *refs):
    if with_cnt:
      agg_out, cnt_out = refs[0], refs[1]
      refs = refs[2:]
    else:
      agg_out = refs[0]
      refs = refs[1:]
    (src_v, dst_v, rows_a, rows_b, ones_v, ztile_v, cbuf_v,
     agg_sh, cnt_sh, sem_a, sem_b) = refs

    c = lax.axis_index("c")
    s = lax.axis_index("s")
    wid = s * NC + c

    zf = jnp.zeros((L,), jnp.float32)
    of = jnp.ones((L,), jnp.float32)
    for r in range(16):
      for g in range(D // L):
        ztile_v[r, pl.ds(g * L, L)] = zf
    if with_cnt:
      for i in range(C // L):
        ones_v[pl.ds(i * L, L)] = of
      for i in range(ROWS_PER_SUB // L):
        cbuf_v[pl.ds(i * L, L)] = zf

    row0 = s * ROWS_PER_SUB

    def gather(g, buf, sem):
      pltpu.async_copy(feat_hbm.at[src_v.at[g]], buf, sem)

    def gwait(buf, sem):
      pltpu.make_async_copy(feat_hbm.at[src_v.at[0]], buf, sem).wait()

    def scat(g, buf):
      pltpu.sync_copy(buf, agg_sh.at[dst_v.at[g]], add=True)
      if with_cnt:
        pltpu.sync_copy(ones_v, cnt_sh.at[dst_v.at[g]], add=True)

    for half in range(2):
      w0 = wid * WPW + half * HALF
      pltpu.sync_copy(edge_hbm.at[0, pl.ds(w0, HALF)], src_v)
      pltpu.sync_copy(edge_hbm.at[1, pl.ds(w0, HALF)], dst_v)
      gather(0, rows_a, sem_a)
      gather(1, rows_b, sem_b)

      if half == 0:
        def zero_step(j, carry):
          pltpu.sync_copy(ztile_v, agg_sh.at[pl.ds(row0 + j * 16, 16)])
          return carry
        lax.fori_loop(0, ROWS_PER_SUB // 16, zero_step, 0)
        if with_cnt:
          pltpu.sync_copy(cbuf_v, cnt_sh.at[pl.ds(row0, ROWS_PER_SUB)])
        plsc.subcore_barrier()

      def pair_step(k, carry):
        g0 = 2 * k
        gwait(rows_a, sem_a)
        scat(g0, rows_a)

        @pl.when(k < HALF // 2 - 1)
        def _():
          gather(g0 + 2, rows_a, sem_a)
        gwait(rows_b, sem_b)
        scat(g0 + 1, rows_b)

        @pl.when(k < HALF // 2 - 1)
        def _():
          gather(g0 + 3, rows_b, sem_b)
        return carry
      lax.fori_loop(0, HALF // 2, pair_step, 0)
    plsc.subcore_barrier()

    pltpu.sync_copy(agg_sh.at[pl.ds(row0, ROWS_PER_SUB)],
                    agg_out.at[c, pl.ds(row0, ROWS_PER_SUB)])
    if with_cnt:
      pltpu.sync_copy(cnt_sh.at[pl.ds(row0, ROWS_PER_SUB)],
                      cnt_out.at[c, pl.ds(row0, ROWS_PER_SUB)])

  return agg


_agg_cnt = _make_agg_kernel(True)
_agg_nocnt = _make_agg_kernel(False)

BR = 2000


def _dot_t(a, w):
  return lax.dot_general(a, w, (((1,), (1,)), ((), ())),
                         preferred_element_type=jnp.float32)


def _self_body(x_ref, wr_ref, b_ref, o_ref):
  o_ref[...] = _dot_t(x_ref[...], wr_ref[...]) + b_ref[...]


def _self_term(x, wr, b):
  return pl.pallas_call(
      _self_body,
      grid=(N // BR,),
      in_specs=[
          pl.BlockSpec((BR, D), lambda i: (i, 0)),
          pl.BlockSpec((D, D), lambda i: (0, 0)),
          pl.BlockSpec((1, D), lambda i: (0, 0)),
      ],
      out_specs=pl.BlockSpec((BR, D), lambda i: (i, 0)),
      out_shape=jax.ShapeDtypeStruct((N, D), jnp.float32),
  )(x, wr, b)


def _combine_body(relu, aggs_ref, cnt_ref, self_ref, wl_ref, o_ref):
  cnt = cnt_ref[0] + cnt_ref[1]
  inv = 1.0 / jnp.maximum(cnt, 1.0)
  mean = (aggs_ref[0] + aggs_ref[1]) * inv
  out = _dot_t(mean, wl_ref[...]) + self_ref[...]
  if relu:
    out = jnp.maximum(out, 0.0)
  o_ref[...] = out


def _combine(aggs, cnts3, selfterm, wl, relu):
  return pl.pallas_call(
      functools.partial(_combine_body, relu),
      grid=(N // BR,),
      in_specs=[
          pl.BlockSpec((NC, BR, D), lambda i: (0, i, 0)),
          pl.BlockSpec((NC, BR, 1), lambda i: (0, i, 0)),
          pl.BlockSpec((BR, D), lambda i: (i, 0)),
          pl.BlockSpec((D, D), lambda i: (0, 0)),
      ],
      out_specs=pl.BlockSpec((BR, D), lambda i: (i, 0)),
      out_shape=jax.ShapeDtypeStruct((N, D), jnp.float32),
  )(aggs, cnts3, selfterm, wl)


def kernel(x, edge_index, W1_l, b1_l, W1_r, W2_l, b2_l, W2_r):
  edge2d = jnp.concatenate(
      [edge_index.astype(jnp.int32), jnp.asarray(_PADS)], axis=1
  ).reshape(2, NWIN, C)

  aggs1, cnts = _agg_cnt(edge2d, x)
  cnts3 = cnts.reshape(NC, NP, 1)
  self1 = _self_term(x, W1_r, b1_l.reshape(1, D))
  h = _combine(aggs1, cnts3, self1, W1_l, True)
  (aggs2,) = _agg_nocnt(edge2d, h)
  self2 = _self_term(h, W2_r, b2_l.reshape(1, D))
  return _combine(aggs2, cnts3, self2, W2_l, False)
